# baseline (speedup 1.0000x reference)
import functools

import jax
import jax.numpy as jnp
from jax import lax
from jax.experimental import pallas as pl
from jax.experimental.pallas import tpu as pltpu
from jax.experimental.pallas import tpu_sc as plsc

B = 16384
ED = 64
FD = 32
HD = 128
NC = 2
NS = 16
NW = NC * NS
RPW = B // NW
CHUNK = 128
NCH = RPW // CHUNK
TB = 2048


def _sc_gather_body(uidx, iidx, f1idx, f2idx, g1idx,
                    user_emb, item_emb, u1t, u2t, i1t,
                    out_ue, out_ie, out_f1, out_f2, out_g1,
                    uiv, iiv, f1v, f2v, g1v,
                    uev, iev, f1r, f2r, g1r,
                    sem0, sem1, sem2, sem3, sem4):
    wid = lax.axis_index("s") * NC + lax.axis_index("c")
    crow = wid * NCH
    base = wid * RPW
    pltpu.sync_copy(uidx.at[pl.ds(crow, NCH)], uiv)
    pltpu.sync_copy(iidx.at[pl.ds(crow, NCH)], iiv)
    pltpu.sync_copy(f1idx.at[pl.ds(crow, NCH)], f1v)
    pltpu.sync_copy(f2idx.at[pl.ds(crow, NCH)], f2v)
    pltpu.sync_copy(g1idx.at[pl.ds(crow, NCH)], g1v)
    handles = []
    for j in range(NCH):
        o = j * CHUNK
        handles.append(pltpu.async_copy(user_emb.at[uiv.at[j]], uev.at[pl.ds(o, CHUNK)], sem0))
        handles.append(pltpu.async_copy(item_emb.at[iiv.at[j]], iev.at[pl.ds(o, CHUNK)], sem1))
        handles.append(pltpu.async_copy(u1t.at[f1v.at[j]], f1r.at[pl.ds(o, CHUNK)], sem2))
        handles.append(pltpu.async_copy(u2t.at[f2v.at[j]], f2r.at[pl.ds(o, CHUNK)], sem3))
        handles.append(pltpu.async_copy(i1t.at[g1v.at[j]], g1r.at[pl.ds(o, CHUNK)], sem4))
    for h in handles:
        h.wait()
    pltpu.sync_copy(uev, out_ue.at[pl.ds(base, RPW)])
    pltpu.sync_copy(iev, out_ie.at[pl.ds(base, RPW)])
    pltpu.sync_copy(f1r, out_f1.at[pl.ds(base, RPW)])
    pltpu.sync_copy(f2r, out_f2.at[pl.ds(base, RPW)])
    pltpu.sync_copy(g1r, out_g1.at[pl.ds(base, RPW)])


_sc_gather = functools.partial(
    pl.kernel,
    mesh=plsc.VectorSubcoreMesh(core_axis_name="c", subcore_axis_name="s"),
    compiler_params=pltpu.CompilerParams(use_tc_tiling_on_sc=False),
    out_type=[
        jax.ShapeDtypeStruct((B, ED), jnp.float32),
        jax.ShapeDtypeStruct((B, ED), jnp.float32),
        jax.ShapeDtypeStruct((B, FD), jnp.float32),
        jax.ShapeDtypeStruct((B, FD), jnp.float32),
        jax.ShapeDtypeStruct((B, FD), jnp.float32),
    ],
    scratch_types=[
        pltpu.VMEM((NCH, CHUNK), jnp.int32),
        pltpu.VMEM((NCH, CHUNK), jnp.int32),
        pltpu.VMEM((NCH, CHUNK), jnp.int32),
        pltpu.VMEM((NCH, CHUNK), jnp.int32),
        pltpu.VMEM((NCH, CHUNK), jnp.int32),
        pltpu.VMEM((RPW, ED), jnp.float32),
        pltpu.VMEM((RPW, ED), jnp.float32),
        pltpu.VMEM((RPW, FD), jnp.float32),
        pltpu.VMEM((RPW, FD), jnp.float32),
        pltpu.VMEM((RPW, FD), jnp.float32),
        pltpu.SemaphoreType.DMA,
        pltpu.SemaphoreType.DMA,
        pltpu.SemaphoreType.DMA,
        pltpu.SemaphoreType.DMA,
        pltpu.SemaphoreType.DMA,
    ],
)(_sc_gather_body)


def _tower_body(ue, ie, f1, f2, g1, uf, itf,
                w1ua, w1ub, w1uc, w1ud, b1u, w2u, b2u,
                w1ia, w1ib, w1ic, b1i, w2i, b2i, out):
    xu = jnp.dot(ue[:], w1ua[:], preferred_element_type=jnp.float32)
    xu += jnp.dot(f1[:], w1ub[:], preferred_element_type=jnp.float32)
    xu += jnp.dot(f2[:], w1uc[:], preferred_element_type=jnp.float32)
    xu += uf[:][:, 2:3] * w1ud[:] + b1u[:]
    xu = jnp.maximum(xu, 0.0)
    ur = jnp.dot(xu, w2u[:], preferred_element_type=jnp.float32) + b2u[:]

    yi = jnp.dot(ie[:], w1ia[:], preferred_element_type=jnp.float32)
    yi += jnp.dot(g1[:], w1ib[:], preferred_element_type=jnp.float32)
    yi += jnp.dot(itf[:][:, 1:18], w1ic[:], preferred_element_type=jnp.float32)
    yi = jnp.maximum(yi + b1i[:], 0.0)
    ir = jnp.dot(yi, w2i[:], preferred_element_type=jnp.float32) + b2i[:]

    out[0, 0, :] = jnp.sum(ur * ir, axis=1)


def kernel(user_ids, item_ids, user_features, item_features, user_emb,
           item_emb, u_cat1, u_cat2, i_cat1, user_fc1_W, user_fc1_b,
           user_fc2_W, user_fc2_b, item_fc1_W, item_fc1_b, item_fc2_W,
           item_fc2_b):
    uidx = user_ids.astype(jnp.int32).reshape(NW * NCH, CHUNK)
    iidx = item_ids.astype(jnp.int32).reshape(NW * NCH, CHUNK)
    f1idx = user_features[:, 0].astype(jnp.int32).reshape(NW * NCH, CHUNK)
    f2idx = user_features[:, 1].astype(jnp.int32).reshape(NW * NCH, CHUNK)
    g1idx = item_features[:, 0].astype(jnp.int32).reshape(NW * NCH, CHUNK)

    ue, ie, f1, f2, g1 = _sc_gather(uidx, iidx, f1idx, f2idx, g1idx,
                                    user_emb, item_emb, u_cat1, u_cat2, i_cat1)

    grid = B // TB
    scores = pl.pallas_call(
        _tower_body,
        grid=(grid,),
        in_specs=[
            pl.BlockSpec((TB, ED), lambda i: (i, 0)),
            pl.BlockSpec((TB, ED), lambda i: (i, 0)),
            pl.BlockSpec((TB, FD), lambda i: (i, 0)),
            pl.BlockSpec((TB, FD), lambda i: (i, 0)),
            pl.BlockSpec((TB, FD), lambda i: (i, 0)),
            pl.BlockSpec((TB, 3), lambda i: (i, 0)),
            pl.BlockSpec((TB, 18), lambda i: (i, 0)),
            pl.BlockSpec((ED, HD), lambda i: (0, 0)),
            pl.BlockSpec((FD, HD), lambda i: (0, 0)),
            pl.BlockSpec((FD, HD), lambda i: (0, 0)),
            pl.BlockSpec((1, HD), lambda i: (0, 0)),
            pl.BlockSpec((1, HD), lambda i: (0, 0)),
            pl.BlockSpec((HD, HD), lambda i: (0, 0)),
            pl.BlockSpec((1, HD), lambda i: (0, 0)),
            pl.BlockSpec((ED, HD), lambda i: (0, 0)),
            pl.BlockSpec((FD, HD), lambda i: (0, 0)),
            pl.BlockSpec((17, HD), lambda i: (0, 0)),
            pl.BlockSpec((1, HD), lambda i: (0, 0)),
            pl.BlockSpec((HD, HD), lambda i: (0, 0)),
            pl.BlockSpec((1, HD), lambda i: (0, 0)),
        ],
        out_specs=pl.BlockSpec((1, 1, TB), lambda i: (i, 0, 0)),
        out_shape=jax.ShapeDtypeStruct((grid, 1, TB), jnp.float32),
    )(
        ue, ie, f1, f2, g1, user_features, item_features,
        user_fc1_W[:ED], user_fc1_W[ED:ED + FD], user_fc1_W[ED + FD:128],
        user_fc1_W[128:129], user_fc1_b.reshape(1, HD),
        user_fc2_W, user_fc2_b.reshape(1, HD),
        item_fc1_W[:ED], item_fc1_W[ED:ED + FD], item_fc1_W[ED + FD:113],
        item_fc1_b.reshape(1, HD),
        item_fc2_W, item_fc2_b.reshape(1, HD),
    )
    return scores.reshape(B)


# speedup vs baseline: 1.3215x; 1.3215x over previous
import functools

import jax
import jax.numpy as jnp
from jax import lax
from jax.experimental import pallas as pl
from jax.experimental.pallas import tpu as pltpu
from jax.experimental.pallas import tpu_sc as plsc

B = 16384
ED = 64
FD = 32
HD = 128
NC = 2
NS = 16
NW = NC * NS
RPW = B // NW
CHUNK = 128
NCH = RPW // CHUNK
RC = 16
NRC = RPW // RC
TB = 2048


def _sc_small_body(f1idx, f2idx, g1idx, u1t, u2t, i1t,
                   out_f1, out_f2, out_g1,
                   f1v, f2v, g1v, f1r, f2r, g1r,
                   sem2, sem3, sem4):
    wid = lax.axis_index("s") * NC + lax.axis_index("c")
    crow = wid * NCH
    base = wid * RPW
    pltpu.sync_copy(f1idx.at[pl.ds(crow, NCH)], f1v)
    pltpu.sync_copy(f2idx.at[pl.ds(crow, NCH)], f2v)
    pltpu.sync_copy(g1idx.at[pl.ds(crow, NCH)], g1v)
    handles = []
    for j in range(NCH):
        o = j * CHUNK
        handles.append(pltpu.async_copy(u1t.at[f1v.at[j]], f1r.at[pl.ds(o, CHUNK)], sem2))
        handles.append(pltpu.async_copy(u2t.at[f2v.at[j]], f2r.at[pl.ds(o, CHUNK)], sem3))
        handles.append(pltpu.async_copy(i1t.at[g1v.at[j]], g1r.at[pl.ds(o, CHUNK)], sem4))
    for h in handles:
        h.wait()
    pltpu.sync_copy(f1r, out_f1.at[pl.ds(base, RPW)])
    pltpu.sync_copy(f2r, out_f2.at[pl.ds(base, RPW)])
    pltpu.sync_copy(g1r, out_g1.at[pl.ds(base, RPW)])


_sc_small = functools.partial(
    pl.kernel,
    mesh=plsc.VectorSubcoreMesh(core_axis_name="c", subcore_axis_name="s"),
    compiler_params=pltpu.CompilerParams(use_tc_tiling_on_sc=False),
    out_type=[
        jax.ShapeDtypeStruct((B, FD), jnp.float32),
        jax.ShapeDtypeStruct((B, FD), jnp.float32),
        jax.ShapeDtypeStruct((B, FD), jnp.float32),
    ],
    scratch_types=[
        pltpu.VMEM((NCH, CHUNK), jnp.int32),
        pltpu.VMEM((NCH, CHUNK), jnp.int32),
        pltpu.VMEM((NCH, CHUNK), jnp.int32),
        pltpu.VMEM((RPW, FD), jnp.float32),
        pltpu.VMEM((RPW, FD), jnp.float32),
        pltpu.VMEM((RPW, FD), jnp.float32),
        pltpu.SemaphoreType.DMA,
        pltpu.SemaphoreType.DMA,
        pltpu.SemaphoreType.DMA,
    ],
)(_sc_small_body)


def _sc_big_body(uidx, iidx, user_emb, item_emb,
                 out_ue, out_ie,
                 uvm, ivm, slab_u, slab_i, row_u, row_i,
                 sem_u, sem_i):
    wid = lax.axis_index("s") * NC + lax.axis_index("c")
    pltpu.sync_copy(uidx.at[wid], uvm)
    pltpu.sync_copy(iidx.at[wid], ivm)

    def chunk(c, carry):
        cbase = pl.multiple_of(c * RC, RC)
        uvec = uvm[pl.ds(cbase, RC)]
        ivec = ivm[pl.ds(cbase, RC)]
        handles = []
        for k in range(RC):
            su = uvec[k]
            si = ivec[k]
            sub = pl.multiple_of((su >> 3) * 8, 8)
            sib = pl.multiple_of((si >> 3) * 8, 8)
            handles.append(pltpu.async_copy(
                user_emb.at[pl.ds(sub, 8)], slab_u.at[pl.ds(k * 8, 8)], sem_u))
            handles.append(pltpu.async_copy(
                item_emb.at[pl.ds(sib, 8)], slab_i.at[pl.ds(k * 8, 8)], sem_i))
        for h in handles:
            h.wait()
        for k in range(RC):
            hu = (uvec[k] & 7) + k * 8
            hi = (ivec[k] & 7) + k * 8
            for j in range(ED // 16):
                row_u[k, pl.ds(j * 16, 16)] = slab_u[hu, pl.ds(j * 16, 16)]
                row_i[k, pl.ds(j * 16, 16)] = slab_i[hi, pl.ds(j * 16, 16)]
        obase = pl.multiple_of(wid * RPW + c * RC, RC)
        pltpu.sync_copy(row_u, out_ue.at[pl.ds(obase, RC)])
        pltpu.sync_copy(row_i, out_ie.at[pl.ds(obase, RC)])
        return carry

    lax.fori_loop(0, NRC, chunk, 0)


_sc_big = functools.partial(
    pl.kernel,
    mesh=plsc.VectorSubcoreMesh(core_axis_name="c", subcore_axis_name="s"),
    out_type=[
        jax.ShapeDtypeStruct((B, ED), jnp.float32),
        jax.ShapeDtypeStruct((B, ED), jnp.float32),
    ],
    scratch_types=[
        pltpu.VMEM((RPW,), jnp.int32),
        pltpu.VMEM((RPW,), jnp.int32),
        pltpu.VMEM((RC * 8, ED), jnp.float32),
        pltpu.VMEM((RC * 8, ED), jnp.float32),
        pltpu.VMEM((RC, ED), jnp.float32),
        pltpu.VMEM((RC, ED), jnp.float32),
        pltpu.SemaphoreType.DMA,
        pltpu.SemaphoreType.DMA,
    ],
)(_sc_big_body)


def _tower_body(ue, ie, f1, f2, g1, uf, itf,
                w1ua, w1ub, w1uc, w1ud, b1u, w2u, b2u,
                w1ia, w1ib, w1ic, b1i, w2i, b2i, out):
    xu = jnp.dot(ue[:], w1ua[:], preferred_element_type=jnp.float32)
    xu += jnp.dot(f1[:], w1ub[:], preferred_element_type=jnp.float32)
    xu += jnp.dot(f2[:], w1uc[:], preferred_element_type=jnp.float32)
    xu += uf[:][:, 2:3] * w1ud[:] + b1u[:]
    xu = jnp.maximum(xu, 0.0)
    ur = jnp.dot(xu, w2u[:], preferred_element_type=jnp.float32) + b2u[:]

    yi = jnp.dot(ie[:], w1ia[:], preferred_element_type=jnp.float32)
    yi += jnp.dot(g1[:], w1ib[:], preferred_element_type=jnp.float32)
    yi += jnp.dot(itf[:][:, 1:18], w1ic[:], preferred_element_type=jnp.float32)
    yi = jnp.maximum(yi + b1i[:], 0.0)
    ir = jnp.dot(yi, w2i[:], preferred_element_type=jnp.float32) + b2i[:]

    out[0, 0, :] = jnp.sum(ur * ir, axis=1)


def kernel(user_ids, item_ids, user_features, item_features, user_emb,
           item_emb, u_cat1, u_cat2, i_cat1, user_fc1_W, user_fc1_b,
           user_fc2_W, user_fc2_b, item_fc1_W, item_fc1_b, item_fc2_W,
           item_fc2_b):
    uidx = user_ids.astype(jnp.int32).reshape(NW, RPW)
    iidx = item_ids.astype(jnp.int32).reshape(NW, RPW)
    f1idx = user_features[:, 0].astype(jnp.int32).reshape(NW * NCH, CHUNK)
    f2idx = user_features[:, 1].astype(jnp.int32).reshape(NW * NCH, CHUNK)
    g1idx = item_features[:, 0].astype(jnp.int32).reshape(NW * NCH, CHUNK)

    f1, f2, g1 = _sc_small(f1idx, f2idx, g1idx, u_cat1, u_cat2, i_cat1)
    ue, ie = _sc_big(uidx, iidx, user_emb, item_emb)

    grid = B // TB
    scores = pl.pallas_call(
        _tower_body,
        grid=(grid,),
        in_specs=[
            pl.BlockSpec((TB, ED), lambda i: (i, 0)),
            pl.BlockSpec((TB, ED), lambda i: (i, 0)),
            pl.BlockSpec((TB, FD), lambda i: (i, 0)),
            pl.BlockSpec((TB, FD), lambda i: (i, 0)),
            pl.BlockSpec((TB, FD), lambda i: (i, 0)),
            pl.BlockSpec((TB, 3), lambda i: (i, 0)),
            pl.BlockSpec((TB, 18), lambda i: (i, 0)),
            pl.BlockSpec((ED, HD), lambda i: (0, 0)),
            pl.BlockSpec((FD, HD), lambda i: (0, 0)),
            pl.BlockSpec((FD, HD), lambda i: (0, 0)),
            pl.BlockSpec((1, HD), lambda i: (0, 0)),
            pl.BlockSpec((1, HD), lambda i: (0, 0)),
            pl.BlockSpec((HD, HD), lambda i: (0, 0)),
            pl.BlockSpec((1, HD), lambda i: (0, 0)),
            pl.BlockSpec((ED, HD), lambda i: (0, 0)),
            pl.BlockSpec((FD, HD), lambda i: (0, 0)),
            pl.BlockSpec((17, HD), lambda i: (0, 0)),
            pl.BlockSpec((1, HD), lambda i: (0, 0)),
            pl.BlockSpec((HD, HD), lambda i: (0, 0)),
            pl.BlockSpec((1, HD), lambda i: (0, 0)),
        ],
        out_specs=pl.BlockSpec((1, 1, TB), lambda i: (i, 0, 0)),
        out_shape=jax.ShapeDtypeStruct((grid, 1, TB), jnp.float32),
    )(
        ue, ie, f1, f2, g1, user_features, item_features,
        user_fc1_W[:ED], user_fc1_W[ED:ED + FD], user_fc1_W[ED + FD:128],
        user_fc1_W[128:129], user_fc1_b.reshape(1, HD),
        user_fc2_W, user_fc2_b.reshape(1, HD),
        item_fc1_W[:ED], item_fc1_W[ED:ED + FD], item_fc1_W[ED + FD:113],
        item_fc1_b.reshape(1, HD),
        item_fc2_W, item_fc2_b.reshape(1, HD),
    )
    return scores.reshape(B)
